# baseline (speedup 1.0000x reference)
import functools

import jax
import jax.numpy as jnp
from jax import lax
from jax.experimental import pallas as pl
from jax.experimental.pallas import tpu as pltpu
from jax.experimental.pallas import tpu_sc as plsc

N = 10000
D = 128
E = 320000
K = 30

NC = 2
NS = 16
EPC = E // NC
EPW = EPC // NS
CH = 80
NIT = EPW // CH
NP = 10240
RPT = NP // NS
ZCH = 64
DegW = 16


def _leaky(x):
    return jnp.where(x >= 0, x, 0.01 * x)


def _seg_body(with_deg, *refs):
    if with_deg:
        (y_hbm, src_hbm, dst_hbm, z_hbm, zd_hbm, ones_hbm, out_hbm, deg_hbm,
         srcb, dstb, rows, stage, ones_v, dstage, acc_sh, dacc_sh,
         sem_g, sem_si, sem_di, sem_s, sem_sd) = refs
    else:
        (y_hbm, src_hbm, dst_hbm, z_hbm, out_hbm,
         srcb, dstb, rows, stage, acc_sh,
         sem_g, sem_si, sem_di, sem_s, sem_sd) = refs

    cid = lax.axis_index("c")
    sid = lax.axis_index("s")

    pltpu.sync_copy(z_hbm, stage)
    if with_deg:
        pltpu.sync_copy(zd_hbm, dstage)
        pltpu.sync_copy(ones_hbm, ones_v)
    for c in range(RPT // ZCH):
        pltpu.sync_copy(stage, acc_sh.at[pl.ds(sid * RPT + c * ZCH, ZCH)])
        if with_deg:
            pltpu.sync_copy(dstage,
                            dacc_sh.at[pl.ds(sid * RPT + c * ZCH, ZCH)])
    plsc.subcore_barrier()

    base0 = cid * EPC + sid * EPW

    pltpu.sync_copy(src_hbm.at[pl.ds(base0, CH)], srcb.at[0])
    pltpu.async_copy(dst_hbm.at[pl.ds(base0, CH)], dstb.at[0], sem_di)
    pltpu.async_copy(y_hbm.at[srcb.at[0]], rows.at[0], sem_g)
    pltpu.async_copy(src_hbm.at[pl.ds(base0 + CH, CH)], srcb.at[1], sem_si)

    def body(it, carry):
        p = jnp.bitwise_and(it, 1)
        q = 1 - p

        pltpu.make_async_copy(y_hbm.at[srcb.at[p]], rows.at[p], sem_g).wait()
        pltpu.make_async_copy(dst_hbm.at[pl.ds(base0, CH)],
                              dstb.at[p], sem_di).wait()

        @pl.when(it + 1 < NIT)
        def _():
            pltpu.make_async_copy(src_hbm.at[pl.ds(base0, CH)],
                                  srcb.at[q], sem_si).wait()
            pltpu.async_copy(y_hbm.at[srcb.at[q]], rows.at[q], sem_g)
            pltpu.async_copy(dst_hbm.at[pl.ds(base0 + (it + 1) * CH, CH)],
                             dstb.at[q], sem_di)

        @pl.when(it + 2 < NIT)
        def _():
            pltpu.async_copy(src_hbm.at[pl.ds(base0 + (it + 2) * CH, CH)],
                             srcb.at[p], sem_si)

        pltpu.sync_copy(rows.at[p], acc_sh.at[dstb.at[p]], add=True)
        if with_deg:
            pltpu.sync_copy(ones_v, dacc_sh.at[dstb.at[p]], add=True)
        return carry

    lax.fori_loop(0, NIT, body, 0)
    plsc.subcore_barrier()

    for c in range(RPT // ZCH):
        o = sid * RPT + c * ZCH
        pltpu.sync_copy(acc_sh.at[pl.ds(o, ZCH)], stage)
        pltpu.sync_copy(stage, out_hbm.at[cid].at[pl.ds(o, ZCH)])
        if with_deg:
            pltpu.sync_copy(dacc_sh.at[pl.ds(o, ZCH)], dstage)
            pltpu.sync_copy(dstage, deg_hbm.at[cid].at[pl.ds(o, ZCH)])


def _make_seg_agg(with_deg):
    mesh = plsc.VectorSubcoreMesh(core_axis_name="c", subcore_axis_name="s",
                                  num_cores=NC, num_subcores=NS)
    out_type = [jax.ShapeDtypeStruct((NC, NP, D), jnp.float32)]
    scratch = [
        pltpu.VMEM((2, CH), jnp.int32),
        pltpu.VMEM((2, CH), jnp.int32),
        pltpu.VMEM((2, CH, D), jnp.float32),
        pltpu.VMEM((ZCH, D), jnp.float32),
    ]
    if with_deg:
        out_type.append(jax.ShapeDtypeStruct((NC, NP, DegW), jnp.float32))
        scratch += [
            pltpu.VMEM((CH, DegW), jnp.float32),
            pltpu.VMEM((ZCH, DegW), jnp.float32),
        ]
    scratch.append(pltpu.VMEM_SHARED((NP, D), jnp.float32))
    if with_deg:
        scratch.append(pltpu.VMEM_SHARED((NP, DegW), jnp.float32))
    scratch += [pltpu.SemaphoreType.DMA] * 5
    return pl.kernel(
        functools.partial(_seg_body, with_deg),
        out_type=out_type,
        mesh=mesh,
        scratch_types=scratch,
        compiler_params=pltpu.CompilerParams(use_tc_tiling_on_sc=False),
    )


_RB = 1000


def _dense_body(y_ref, a_ref, dp_ref, ws_ref, wn_ref, b_ref, o_ref):
    deg = dp_ref[0, :, 0:1] + dp_ref[1, :, 0:1]
    inv = 1.0 / jnp.maximum(deg, 1.0)
    mean = (a_ref[0] + a_ref[1]) * inv
    acc = jnp.dot(y_ref[...], ws_ref[...], preferred_element_type=jnp.float32)
    acc += jnp.dot(mean, wn_ref[...], preferred_element_type=jnp.float32)
    o_ref[...] = _leaky(acc + b_ref[...])


def _dense_layer(y, agg, degp, Ws, Wn, b):
    return pl.pallas_call(
        _dense_body,
        grid=(N // _RB,),
        in_specs=[
            pl.BlockSpec((_RB, D), lambda i: (i, 0)),
            pl.BlockSpec((NC, _RB, D), lambda i: (0, i, 0)),
            pl.BlockSpec((NC, _RB, DegW), lambda i: (0, i, 0)),
            pl.BlockSpec((D, D), lambda i: (0, 0)),
            pl.BlockSpec((D, D), lambda i: (0, 0)),
            pl.BlockSpec((1, D), lambda i: (0, 0)),
        ],
        out_specs=pl.BlockSpec((_RB, D), lambda i: (i, 0)),
        out_shape=jax.ShapeDtypeStruct((N, D), jnp.float32),
    )(y, agg, degp, Ws, Wn, b.reshape(1, D))


def _pool_body(y_ref, o_ref):
    ri = lax.broadcasted_iota(jnp.int32, (D, D), 0)
    ci = lax.broadcasted_iota(jnp.int32, (D, D), 1)
    eye = (ri == ci).astype(jnp.float32)
    cif = ci.astype(jnp.float32)
    keys0 = jnp.max(y_ref[...], axis=1, keepdims=True)
    node_iota = lax.broadcasted_iota(jnp.int32, (N, 1), 0)

    def body(i, keys):
        m = jnp.max(keys)
        sel = keys == m
        j = jnp.min(jnp.where(sel, node_iota, N))
        row = y_ref[pl.ds(j, 1), :]
        vcol = lax.dot_general(eye, row, (((1,), (1,)), ((), ())),
                               preferred_element_type=jnp.float32)
        less = (row < vcol).astype(jnp.float32)
        ties = ((row == vcol) & (ci < ri)).astype(jnp.float32)
        rank = jnp.sum(less + ties, axis=1, keepdims=True)
        onehot = (rank == cif).astype(jnp.float32)
        o_ref[pl.ds(i, 1), :] = jnp.sum(vcol * onehot, axis=0, keepdims=True)
        return jnp.where(node_iota == j, -jnp.inf, keys)

    lax.fori_loop(0, K, body, keys0)


def _sort_pool(y):
    return pl.pallas_call(
        _pool_body,
        out_shape=jax.ShapeDtypeStruct((K, D), jnp.float32),
    )(y)


_CL = K * D - 4
_CP = K * D


def _head_body(p5_ref, w1_ref, cw_ref, cb_ref, b1_ref, w2_ref, b2_ref, o_ref,
               acc_ref):
    o = pl.program_id(0)

    @pl.when(o == 0)
    def _():
        acc_ref[...] = b1_ref[...]

    p5 = p5_ref[...][:, :_CL]
    blk = w1_ref[0]
    part = jnp.dot(p5, blk, preferred_element_type=jnp.float32)
    sel8 = (lax.broadcasted_iota(jnp.int32, (8, 1), 0) == o).astype(jnp.float32)
    wrow = jnp.sum(cw_ref[...] * sel8, axis=0, keepdims=True)
    cb = jnp.sum(cb_ref[...] * sel8, axis=0, keepdims=True)
    acc_ref[...] += jnp.dot(wrow, part, preferred_element_type=jnp.float32)
    acc_ref[...] += cb * jnp.sum(blk, axis=0, keepdims=True)

    @pl.when(o == 7)
    def _():
        h = _leaky(acc_ref[...])
        o_ref[...] = (jnp.dot(h, w2_ref[...], preferred_element_type=jnp.float32)
                      + b2_ref[...])


def _head(p5, w1r, cw, cb, b1, w2p, b2p):
    return pl.pallas_call(
        _head_body,
        grid=(8,),
        in_specs=[
            pl.BlockSpec((5, _CP), lambda o: (0, 0)),
            pl.BlockSpec((1, _CL, D), lambda o: (o, 0, 0)),
            pl.BlockSpec((8, 5), lambda o: (0, 0)),
            pl.BlockSpec((8, 1), lambda o: (0, 0)),
            pl.BlockSpec((1, D), lambda o: (0, 0)),
            pl.BlockSpec((D, D), lambda o: (0, 0)),
            pl.BlockSpec((1, D), lambda o: (0, 0)),
        ],
        out_specs=pl.BlockSpec((1, D), lambda o: (0, 0)),
        out_shape=jax.ShapeDtypeStruct((1, D), jnp.float32),
        scratch_shapes=[pltpu.VMEM((1, D), jnp.float32)],
    )(p5, w1r, cw, cb, b1, w2p, b2p)


def kernel(x, edge_index, W_self_0, W_neigh_0, b_0, W_self_1, W_neigh_1, b_1,
           W_self_2, W_neigh_2, b_2, conv_w, conv_b, W_fc1, b_fc1, W_fc2, b_fc2):
    src = edge_index[0]
    dst = edge_index[1]
    z = jnp.zeros((ZCH, D), jnp.float32)
    zd = jnp.zeros((ZCH, DegW), jnp.float32)
    ones = jnp.ones((CH, DegW), jnp.float32)

    agg0, degp = _make_seg_agg(True)(x, src, dst, z, zd, ones)
    y = _dense_layer(x, agg0, degp, W_self_0, W_neigh_0, b_0)
    agg1, = _make_seg_agg(False)(y, src, dst, z)
    y = _dense_layer(y, agg1, degp, W_self_1, W_neigh_1, b_1)
    agg2, = _make_seg_agg(False)(y, src, dst, z)
    y = _dense_layer(y, agg2, degp, W_self_2, W_neigh_2, b_2)

    pooled = _sort_pool(y)

    flat = pooled.reshape(-1)
    flatp = jnp.concatenate([flat, jnp.zeros((4,), jnp.float32)])
    p5 = jnp.stack([lax.slice(flatp, (k,), (k + _CP,)) for k in range(5)])
    w1r = W_fc1.reshape(8, _CL, D)
    w2p = jnp.pad(W_fc2, ((0, 0), (0, D - 10)))
    b2p = jnp.pad(b_fc2, (0, D - 10)).reshape(1, D)
    out = _head(p5, w1r, conv_w[:, 0, :], conv_b.reshape(8, 1),
                b_fc1.reshape(1, D), w2p, b2p)
    return out[:, :10]


# speedup vs baseline: 1.1957x; 1.1957x over previous
import functools

import jax
import jax.numpy as jnp
from jax import lax
from jax.experimental import pallas as pl
from jax.experimental.pallas import tpu as pltpu
from jax.experimental.pallas import tpu_sc as plsc

N = 10000
D = 128
E = 320000
K = 30

NC = 2
NS = 16
EPC = E // NC
EPW = EPC // NS
CH = 80
NIT = EPW // CH
NP = 10240
RPT = NP // NS
ZCH = 64
DegW = 16


def _leaky(x):
    return jnp.where(x >= 0, x, 0.01 * x)


def _seg_body(with_deg, *refs):
    if with_deg:
        (y_hbm, src_hbm, dst_hbm, z_hbm, zd_hbm, ones_hbm, out_hbm, deg_hbm,
         srcb, dstb, rows, stage, ones_v, dstage, acc_sh, dacc_sh,
         sem_g, sem_si, sem_di, sem_s, sem_sd) = refs
    else:
        (y_hbm, src_hbm, dst_hbm, z_hbm, out_hbm,
         srcb, dstb, rows, stage, acc_sh,
         sem_g, sem_si, sem_di, sem_s, sem_sd) = refs

    cid = lax.axis_index("c")
    sid = lax.axis_index("s")

    pltpu.sync_copy(z_hbm, stage)
    if with_deg:
        pltpu.sync_copy(zd_hbm, dstage)
        pltpu.sync_copy(ones_hbm, ones_v)
    for c in range(RPT // ZCH):
        pltpu.sync_copy(stage, acc_sh.at[pl.ds(sid * RPT + c * ZCH, ZCH)])
        if with_deg:
            pltpu.sync_copy(dstage,
                            dacc_sh.at[pl.ds(sid * RPT + c * ZCH, ZCH)])
    plsc.subcore_barrier()

    base0 = cid * EPC + sid * EPW

    pltpu.sync_copy(src_hbm.at[pl.ds(base0, CH)], srcb.at[0])
    pltpu.async_copy(dst_hbm.at[pl.ds(base0, CH)], dstb.at[0], sem_di.at[0])
    pltpu.async_copy(y_hbm.at[srcb.at[0]], rows.at[0], sem_g.at[0])
    pltpu.async_copy(src_hbm.at[pl.ds(base0 + CH, CH)], srcb.at[1],
                     sem_si.at[1])

    def body(it, carry):
        p = jnp.bitwise_and(it, 1)
        q = 1 - p

        @pl.when((it >= 1) & (it + 1 < NIT))
        def _():
            pltpu.make_async_copy(rows.at[q], acc_sh.at[dstb.at[q]],
                                  sem_s.at[q]).wait()
            if with_deg:
                pltpu.make_async_copy(ones_v, dacc_sh.at[dstb.at[q]],
                                      sem_sd.at[q]).wait()

        @pl.when(it + 1 < NIT)
        def _():
            pltpu.make_async_copy(src_hbm.at[pl.ds(base0, CH)],
                                  srcb.at[q], sem_si.at[q]).wait()
            pltpu.async_copy(y_hbm.at[srcb.at[q]], rows.at[q], sem_g.at[q])
            pltpu.async_copy(dst_hbm.at[pl.ds(base0 + (it + 1) * CH, CH)],
                             dstb.at[q], sem_di.at[q])

        @pl.when(it + 2 < NIT)
        def _():
            pltpu.async_copy(src_hbm.at[pl.ds(base0 + (it + 2) * CH, CH)],
                             srcb.at[p], sem_si.at[p])

        pltpu.make_async_copy(y_hbm.at[srcb.at[p]], rows.at[p],
                              sem_g.at[p]).wait()
        pltpu.make_async_copy(dst_hbm.at[pl.ds(base0, CH)],
                              dstb.at[p], sem_di.at[p]).wait()
        pltpu.async_copy(rows.at[p], acc_sh.at[dstb.at[p]], sem_s.at[p],
                         add=True)
        if with_deg:
            pltpu.async_copy(ones_v, dacc_sh.at[dstb.at[p]], sem_sd.at[p],
                             add=True)
        return carry

    lax.fori_loop(0, NIT, body, 0)
    for t in range(2):
        pltpu.make_async_copy(rows.at[t], acc_sh.at[dstb.at[t]],
                              sem_s.at[t]).wait()
        if with_deg:
            pltpu.make_async_copy(ones_v, dacc_sh.at[dstb.at[t]],
                                  sem_sd.at[t]).wait()
    plsc.subcore_barrier()

    for c in range(RPT // ZCH):
        o = sid * RPT + c * ZCH
        pltpu.sync_copy(acc_sh.at[pl.ds(o, ZCH)], stage)
        pltpu.sync_copy(stage, out_hbm.at[cid].at[pl.ds(o, ZCH)])
        if with_deg:
            pltpu.sync_copy(dacc_sh.at[pl.ds(o, ZCH)], dstage)
            pltpu.sync_copy(dstage, deg_hbm.at[cid].at[pl.ds(o, ZCH)])


def _make_seg_agg(with_deg):
    mesh = plsc.VectorSubcoreMesh(core_axis_name="c", subcore_axis_name="s",
                                  num_cores=NC, num_subcores=NS)
    out_type = [jax.ShapeDtypeStruct((NC, NP, D), jnp.float32)]
    scratch = [
        pltpu.VMEM((2, CH), jnp.int32),
        pltpu.VMEM((2, CH), jnp.int32),
        pltpu.VMEM((2, CH, D), jnp.float32),
        pltpu.VMEM((ZCH, D), jnp.float32),
    ]
    if with_deg:
        out_type.append(jax.ShapeDtypeStruct((NC, NP, DegW), jnp.float32))
        scratch += [
            pltpu.VMEM((CH, DegW), jnp.float32),
            pltpu.VMEM((ZCH, DegW), jnp.float32),
        ]
    scratch.append(pltpu.VMEM_SHARED((NP, D), jnp.float32))
    if with_deg:
        scratch.append(pltpu.VMEM_SHARED((NP, DegW), jnp.float32))
    scratch += [pltpu.SemaphoreType.DMA((2,))] * 5
    return pl.kernel(
        functools.partial(_seg_body, with_deg),
        out_type=out_type,
        mesh=mesh,
        scratch_types=scratch,
        compiler_params=pltpu.CompilerParams(use_tc_tiling_on_sc=False),
    )


_RB = 1000


def _dense_body(y_ref, a_ref, dp_ref, ws_ref, wn_ref, b_ref, o_ref):
    deg = dp_ref[0, :, 0:1] + dp_ref[1, :, 0:1]
    inv = 1.0 / jnp.maximum(deg, 1.0)
    mean = (a_ref[0] + a_ref[1]) * inv
    acc = jnp.dot(y_ref[...], ws_ref[...], preferred_element_type=jnp.float32)
    acc += jnp.dot(mean, wn_ref[...], preferred_element_type=jnp.float32)
    o_ref[...] = _leaky(acc + b_ref[...])


def _dense_layer(y, agg, degp, Ws, Wn, b):
    return pl.pallas_call(
        _dense_body,
        grid=(N // _RB,),
        in_specs=[
            pl.BlockSpec((_RB, D), lambda i: (i, 0)),
            pl.BlockSpec((NC, _RB, D), lambda i: (0, i, 0)),
            pl.BlockSpec((NC, _RB, DegW), lambda i: (0, i, 0)),
            pl.BlockSpec((D, D), lambda i: (0, 0)),
            pl.BlockSpec((D, D), lambda i: (0, 0)),
            pl.BlockSpec((1, D), lambda i: (0, 0)),
        ],
        out_specs=pl.BlockSpec((_RB, D), lambda i: (i, 0)),
        out_shape=jax.ShapeDtypeStruct((N, D), jnp.float32),
    )(y, agg, degp, Ws, Wn, b.reshape(1, D))


def _pool_body(y_ref, o_ref):
    ri = lax.broadcasted_iota(jnp.int32, (D, D), 0)
    ci = lax.broadcasted_iota(jnp.int32, (D, D), 1)
    eye = (ri == ci).astype(jnp.float32)
    cif = ci.astype(jnp.float32)
    keys0 = jnp.max(y_ref[...], axis=1, keepdims=True)
    node_iota = lax.broadcasted_iota(jnp.int32, (N, 1), 0)

    def body(i, keys):
        m = jnp.max(keys)
        sel = keys == m
        j = jnp.min(jnp.where(sel, node_iota, N))
        row = y_ref[pl.ds(j, 1), :]
        vcol = lax.dot_general(eye, row, (((1,), (1,)), ((), ())),
                               preferred_element_type=jnp.float32)
        less = (row < vcol).astype(jnp.float32)
        ties = ((row == vcol) & (ci < ri)).astype(jnp.float32)
        rank = jnp.sum(less + ties, axis=1, keepdims=True)
        onehot = (rank == cif).astype(jnp.float32)
        o_ref[pl.ds(i, 1), :] = jnp.sum(vcol * onehot, axis=0, keepdims=True)
        return jnp.where(node_iota == j, -jnp.inf, keys)

    lax.fori_loop(0, K, body, keys0)


def _sort_pool(y):
    return pl.pallas_call(
        _pool_body,
        out_shape=jax.ShapeDtypeStruct((K, D), jnp.float32),
    )(y)


_CL = K * D - 4
_CP = K * D


def _head_body(p5_ref, w1_ref, cw_ref, cb_ref, b1_ref, w2_ref, b2_ref, o_ref,
               acc_ref):
    o = pl.program_id(0)

    @pl.when(o == 0)
    def _():
        acc_ref[...] = b1_ref[...]

    p5 = p5_ref[...][:, :_CL]
    blk = w1_ref[0]
    part = jnp.dot(p5, blk, preferred_element_type=jnp.float32)
    sel8 = (lax.broadcasted_iota(jnp.int32, (8, 1), 0) == o).astype(jnp.float32)
    wrow = jnp.sum(cw_ref[...] * sel8, axis=0, keepdims=True)
    cb = jnp.sum(cb_ref[...] * sel8, axis=0, keepdims=True)
    acc_ref[...] += jnp.dot(wrow, part, preferred_element_type=jnp.float32)
    acc_ref[...] += cb * jnp.sum(blk, axis=0, keepdims=True)

    @pl.when(o == 7)
    def _():
        h = _leaky(acc_ref[...])
        o_ref[...] = (jnp.dot(h, w2_ref[...], preferred_element_type=jnp.float32)
                      + b2_ref[...])


def _head(p5, w1r, cw, cb, b1, w2p, b2p):
    return pl.pallas_call(
        _head_body,
        grid=(8,),
        in_specs=[
            pl.BlockSpec((5, _CP), lambda o: (0, 0)),
            pl.BlockSpec((1, _CL, D), lambda o: (o, 0, 0)),
            pl.BlockSpec((8, 5), lambda o: (0, 0)),
            pl.BlockSpec((8, 1), lambda o: (0, 0)),
            pl.BlockSpec((1, D), lambda o: (0, 0)),
            pl.BlockSpec((D, D), lambda o: (0, 0)),
            pl.BlockSpec((1, D), lambda o: (0, 0)),
        ],
        out_specs=pl.BlockSpec((1, D), lambda o: (0, 0)),
        out_shape=jax.ShapeDtypeStruct((1, D), jnp.float32),
        scratch_shapes=[pltpu.VMEM((1, D), jnp.float32)],
    )(p5, w1r, cw, cb, b1, w2p, b2p)


def kernel(x, edge_index, W_self_0, W_neigh_0, b_0, W_self_1, W_neigh_1, b_1,
           W_self_2, W_neigh_2, b_2, conv_w, conv_b, W_fc1, b_fc1, W_fc2, b_fc2):
    src = edge_index[0]
    dst = edge_index[1]
    z = jnp.zeros((ZCH, D), jnp.float32)
    zd = jnp.zeros((ZCH, DegW), jnp.float32)
    ones = jnp.ones((CH, DegW), jnp.float32)

    agg0, degp = _make_seg_agg(True)(x, src, dst, z, zd, ones)
    y = _dense_layer(x, agg0, degp, W_self_0, W_neigh_0, b_0)
    agg1, = _make_seg_agg(False)(y, src, dst, z)
    y = _dense_layer(y, agg1, degp, W_self_1, W_neigh_1, b_1)
    agg2, = _make_seg_agg(False)(y, src, dst, z)
    y = _dense_layer(y, agg2, degp, W_self_2, W_neigh_2, b_2)

    pooled = _sort_pool(y)

    flat = pooled.reshape(-1)
    flatp = jnp.concatenate([flat, jnp.zeros((4,), jnp.float32)])
    p5 = jnp.stack([lax.slice(flatp, (k,), (k + _CP,)) for k in range(5)])
    w1r = W_fc1.reshape(8, _CL, D)
    w2p = jnp.pad(W_fc2, ((0, 0), (0, D - 10)))
    b2p = jnp.pad(b_fc2, (0, D - 10)).reshape(1, D)
    out = _head(p5, w1r, conv_w[:, 0, :], conv_b.reshape(8, 1),
                b_fc1.reshape(1, D), w2p, b2p)
    return out[:, :10]
